# baseline (speedup 1.0000x reference)
import jax
import jax.numpy as jnp
from jax import lax
from jax.experimental import pallas as pl

_B = 16384
_C = 1000
_BC = 4096


def _onehot_block(ids_ref, o_ref):
    ids = ids_ref[0]
    in_vocab = (ids >= 0) & (ids < _C)
    mapped = jnp.where(in_vocab, ids, _C - 1)
    row = lax.broadcasted_iota(jnp.int32, (_C, _BC), 0)
    o_ref[...] = (row == mapped).astype(jnp.float32)


def kernel(user_ids):
    ids = user_ids.astype(jnp.int32).reshape(_B // _BC, 1, _BC)
    out_t = pl.pallas_call(
        _onehot_block,
        grid=(_B // _BC,),
        in_specs=[pl.BlockSpec((1, 1, _BC), lambda j: (j, 0, 0))],
        out_specs=pl.BlockSpec((_C, _BC), lambda j: (0, j)),
        out_shape=jax.ShapeDtypeStruct((_C, _B), jnp.float32),
    )(ids)
    return out_t.T


# speedup vs baseline: 1.1108x; 1.1108x over previous
import jax
import jax.numpy as jnp
from jax import lax
from jax.experimental import pallas as pl

_B = 16384
_C = 1000
_BC = 1024


def _onehot_block(ids_ref, o_ref):
    ids = ids_ref[0]
    in_vocab = (ids >= 0) & (ids < _C)
    mapped = jnp.where(in_vocab, ids, _C - 1)
    row = lax.broadcasted_iota(jnp.int32, (_C, _BC), 0)
    o_ref[...] = (row == mapped).astype(jnp.float32)


def kernel(user_ids):
    ids = user_ids.astype(jnp.int32).reshape(_B // _BC, 1, _BC)
    out_t = pl.pallas_call(
        _onehot_block,
        grid=(_B // _BC,),
        in_specs=[pl.BlockSpec((1, 1, _BC), lambda j: (j, 0, 0))],
        out_specs=pl.BlockSpec((_C, _BC), lambda j: (0, j)),
        out_shape=jax.ShapeDtypeStruct((_C, _B), jnp.float32),
    )(ids)
    return out_t.T
